# baseline (speedup 1.0000x reference)
import functools

import numpy as np

import jax
import jax.numpy as jnp
from jax.experimental import pallas as pl
from jax.experimental.pallas import tpu as pltpu


_VMEM_LIMIT = 48 * 1024 * 1024


def _hann_padded(win_length, n_fft):
    n = np.arange(win_length)
    w = 0.5 - 0.5 * np.cos(2.0 * np.pi * n / win_length)
    out = np.zeros(n_fft, dtype=np.float64)
    lpad = (n_fft - win_length) // 2
    out[lpad:lpad + win_length] = w
    return out


@functools.lru_cache(maxsize=None)
def _host_tables(n_fft, win_length, hop, T, p0, nb_out):
    F = n_fft // 2 + 1
    win = _hann_padded(win_length, n_fft)
    f = np.arange(F, dtype=np.float64)[:, None]
    o = np.arange(n_fft, dtype=np.float64)[None, :]
    ang = (2.0 * np.pi / n_fft) * f * o
    dup = np.ones((F, 1))
    dup[1:F - 1] = 2.0
    scale = win[None, :] / n_fft
    A = (dup * np.cos(ang)) * scale
    Bm = (-dup * np.sin(ang)) * scale

    win_sq = win ** 2
    n_samples = (T - 1) * hop + n_fft
    wsum = np.zeros(n_samples, dtype=np.float64)
    for t in range(T):
        wsum[t * hop:t * hop + n_fft] += win_sq
    inv = 1.0 / np.clip(wsum, 1e-11, None)
    inv_blocks = inv.reshape(-1, hop)[p0:p0 + nb_out].astype(np.float32)

    to_bf16 = lambda m: jnp.asarray(m.astype(np.float32), dtype=jnp.bfloat16)
    return to_bf16(A), to_bf16(Bm), jnp.asarray(inv_blocks)


def _fused_kernel(re_ref, im_ref, a_ref, b_ref, inv_ref, o_ref, acc_ref,
                  *, ratio, T, p0, nb_out, hop, gb):
    fr = jnp.dot(re_ref[0], a_ref[...], preferred_element_type=jnp.float32)
    fr = fr + jnp.dot(im_ref[0], b_ref[...], preferred_element_type=jnp.float32)
    for j in range(gb):
        rows = fr[j * T:(j + 1) * T, :]
        acc_ref[j, 0:T, :] = rows[:, 0:hop]
        acc_ref[j, T:, :] = jnp.zeros((ratio - 1, hop), jnp.float32)
        for k in range(1, ratio):
            acc_ref[j, k:k + T, :] += rows[:, k * hop:(k + 1) * hop]
        o_ref[0, j] = acc_ref[j, p0:p0 + nb_out, :] * inv_ref[...]


def _fused_istft(re4, im4, *, n_fft, hop, length):
    B, C, T, F = re4.shape
    assert C == 1 and F == n_fft // 2 + 1
    re = re4[:, 0].astype(jnp.bfloat16)
    im = im4[:, 0].astype(jnp.bfloat16)
    ratio = n_fft // hop
    start = n_fft // 2
    assert start % hop == 0 and length % hop == 0
    p0 = start // hop
    nb_out = length // hop
    A, Bm, inv_blocks = _host_tables(n_fft, n_fft, hop, T, p0, nb_out)

    gb = 2 if B % 2 == 0 else 1
    G = B // gb
    re = re.reshape(G, gb * T, F)
    im = im.reshape(G, gb * T, F)

    body = functools.partial(_fused_kernel, ratio=ratio, T=T, p0=p0,
                             nb_out=nb_out, hop=hop, gb=gb)

    def call(re_s, im_s):
        Gs = re_s.shape[0]
        return pl.pallas_call(
            body,
            out_shape=jax.ShapeDtypeStruct((Gs, gb, nb_out, hop), jnp.float32),
            grid=(Gs,),
            in_specs=[
                pl.BlockSpec((1, gb * T, F), lambda g: (g, 0, 0)),
                pl.BlockSpec((1, gb * T, F), lambda g: (g, 0, 0)),
                pl.BlockSpec((F, n_fft), lambda g: (0, 0)),
                pl.BlockSpec((F, n_fft), lambda g: (0, 0)),
                pl.BlockSpec((nb_out, hop), lambda g: (0, 0)),
            ],
            out_specs=pl.BlockSpec((1, gb, nb_out, hop),
                                   lambda g: (g, 0, 0, 0)),
            scratch_shapes=[pltpu.VMEM((gb, T + ratio - 1, hop),
                                       jnp.float32)],
            compiler_params=pltpu.CompilerParams(
                dimension_semantics=("parallel",),
                vmem_limit_bytes=_VMEM_LIMIT,
            ),
        )(re_s, im_s, A, Bm, inv_blocks)

    n_split = 1
    Gc = G // n_split
    ys = [call(re[i * Gc:(i + 1) * Gc], im[i * Gc:(i + 1) * Gc])
          for i in range(n_split)]
    y = jnp.concatenate(ys, axis=0) if n_split > 1 else ys[0]
    return y.reshape(B, length)


def kernel(real_stft, imag_stft):
    return _fused_istft(real_stft, imag_stft,
                        n_fft=2048, hop=512, length=261632)


# speedup vs baseline: 1.8482x; 1.8482x over previous
import functools

import numpy as np

import jax
import jax.numpy as jnp
from jax.experimental import pallas as pl
from jax.experimental.pallas import tpu as pltpu


_VMEM_LIMIT = 60 * 1024 * 1024


def _hann_padded(win_length, n_fft):
    n = np.arange(win_length)
    w = 0.5 - 0.5 * np.cos(2.0 * np.pi * n / win_length)
    out = np.zeros(n_fft, dtype=np.float64)
    lpad = (n_fft - win_length) // 2
    out[lpad:lpad + win_length] = w
    return out


@functools.lru_cache(maxsize=None)
def _host_tables(n_fft, win_length, hop, T, p0, nb_out):
    F = n_fft // 2 + 1
    win = _hann_padded(win_length, n_fft)
    f = np.arange(F, dtype=np.float64)[:, None]
    o = np.arange(n_fft, dtype=np.float64)[None, :]
    ang = (2.0 * np.pi / n_fft) * f * o
    dup = np.ones((F, 1))
    dup[1:F - 1] = 2.0
    scale = win[None, :] / n_fft
    A = (dup * np.cos(ang)) * scale
    Bm = (-dup * np.sin(ang)) * scale

    win_sq = win ** 2
    n_samples = (T - 1) * hop + n_fft
    wsum = np.zeros(n_samples, dtype=np.float64)
    for t in range(T):
        wsum[t * hop:t * hop + n_fft] += win_sq
    inv = 1.0 / np.clip(wsum, 1e-11, None)
    inv_blocks = inv.reshape(-1, hop)[p0:p0 + nb_out].astype(np.float32)

    to_bf16 = lambda m: jnp.asarray(m.astype(np.float32), dtype=jnp.bfloat16)
    return to_bf16(A), to_bf16(Bm), jnp.asarray(inv_blocks)


def _fused_kernel(re_ref, im_ref, a_ref, b_ref, inv_ref, o_ref, acc_ref,
                  *, ratio, T, p0, nb_out, hop, gb):
    fr = jnp.dot(re_ref[0].astype(jnp.bfloat16), a_ref[...],
                 preferred_element_type=jnp.float32)
    fr = fr + jnp.dot(im_ref[0].astype(jnp.bfloat16), b_ref[...],
                      preferred_element_type=jnp.float32)
    for j in range(gb):
        rows = fr[j * T:(j + 1) * T, :]
        acc_ref[j, 0:T, :] = rows[:, 0:hop]
        acc_ref[j, T:, :] = jnp.zeros((ratio - 1, hop), jnp.float32)
        for k in range(1, ratio):
            acc_ref[j, k:k + T, :] += rows[:, k * hop:(k + 1) * hop]
        o_ref[0, j] = acc_ref[j, p0:p0 + nb_out, :] * inv_ref[...]


def _fused_istft(re4, im4, *, n_fft, hop, length):
    B, C, T, F = re4.shape
    assert C == 1 and F == n_fft // 2 + 1
    re = re4[:, 0]
    im = im4[:, 0]
    ratio = n_fft // hop
    start = n_fft // 2
    assert start % hop == 0 and length % hop == 0
    p0 = start // hop
    nb_out = length // hop
    A, Bm, inv_blocks = _host_tables(n_fft, n_fft, hop, T, p0, nb_out)

    gb = 2 if B % 2 == 0 else 1
    G = B // gb
    re = re.reshape(G, gb * T, F)
    im = im.reshape(G, gb * T, F)

    body = functools.partial(_fused_kernel, ratio=ratio, T=T, p0=p0,
                             nb_out=nb_out, hop=hop, gb=gb)

    def call(re_s, im_s):
        Gs = re_s.shape[0]
        return pl.pallas_call(
            body,
            out_shape=jax.ShapeDtypeStruct((Gs, gb, nb_out, hop), jnp.float32),
            grid=(Gs,),
            in_specs=[
                pl.BlockSpec((1, gb * T, F), lambda g: (g, 0, 0)),
                pl.BlockSpec((1, gb * T, F), lambda g: (g, 0, 0)),
                pl.BlockSpec((F, n_fft), lambda g: (0, 0)),
                pl.BlockSpec((F, n_fft), lambda g: (0, 0)),
                pl.BlockSpec((nb_out, hop), lambda g: (0, 0)),
            ],
            out_specs=pl.BlockSpec((1, gb, nb_out, hop),
                                   lambda g: (g, 0, 0, 0)),
            scratch_shapes=[pltpu.VMEM((gb, T + ratio - 1, hop),
                                       jnp.float32)],
            compiler_params=pltpu.CompilerParams(
                dimension_semantics=("parallel",),
                vmem_limit_bytes=_VMEM_LIMIT,
            ),
        )(re_s, im_s, A, Bm, inv_blocks)

    n_split = 1
    Gc = G // n_split
    ys = [call(re[i * Gc:(i + 1) * Gc], im[i * Gc:(i + 1) * Gc])
          for i in range(n_split)]
    y = jnp.concatenate(ys, axis=0) if n_split > 1 else ys[0]
    return y.reshape(B, length)


def kernel(real_stft, imag_stft):
    return _fused_istft(real_stft, imag_stft,
                        n_fft=2048, hop=512, length=261632)


# speedup vs baseline: 2.0222x; 1.0941x over previous
import functools

import numpy as np

import jax
import jax.numpy as jnp
from jax.experimental import pallas as pl
from jax.experimental.pallas import tpu as pltpu


_VMEM_LIMIT = 60 * 1024 * 1024


def _hann_padded(win_length, n_fft):
    n = np.arange(win_length)
    w = 0.5 - 0.5 * np.cos(2.0 * np.pi * n / win_length)
    out = np.zeros(n_fft, dtype=np.float64)
    lpad = (n_fft - win_length) // 2
    out[lpad:lpad + win_length] = w
    return out


@functools.lru_cache(maxsize=None)
def _host_tables(n_fft, win_length, hop, T, p0, nb_out):
    F = n_fft // 2 + 1
    win = _hann_padded(win_length, n_fft)
    f = np.arange(F, dtype=np.float64)[:, None]
    o = np.arange(n_fft, dtype=np.float64)[None, :]
    ang = (2.0 * np.pi / n_fft) * f * o
    dup = np.ones((F, 1))
    dup[1:F - 1] = 2.0
    scale = win[None, :] / n_fft
    A = (dup * np.cos(ang)) * scale
    Bm = (-dup * np.sin(ang)) * scale

    win_sq = win ** 2
    n_samples = (T - 1) * hop + n_fft
    wsum = np.zeros(n_samples, dtype=np.float64)
    for t in range(T):
        wsum[t * hop:t * hop + n_fft] += win_sq
    inv = 1.0 / np.clip(wsum, 1e-11, None)
    inv_blocks = inv.reshape(-1, hop)[p0:p0 + nb_out].astype(np.float32)

    assert np.max(np.abs(Bm[F - 1])) < 1e-12
    nyq = A[F - 1:F].astype(np.float32)
    to_bf16 = lambda m: jnp.asarray(m.astype(np.float32), dtype=jnp.bfloat16)
    return (to_bf16(A[:F - 1]), to_bf16(Bm[:F - 1]),
            jnp.asarray(nyq), jnp.asarray(inv_blocks))


def _fused_kernel(re_ref, im_ref, a_ref, b_ref, nyq_ref, inv_ref, o_ref,
                  acc_ref, *, ratio, T, p0, nb_out, hop, gb):
    Fm = a_ref.shape[0]
    fr = jnp.dot(re_ref[0, :, :Fm].astype(jnp.bfloat16), a_ref[...],
                 preferred_element_type=jnp.float32)
    fr = fr + jnp.dot(im_ref[0, :, :Fm].astype(jnp.bfloat16), b_ref[...],
                      preferred_element_type=jnp.float32)
    fr = fr + re_ref[0, :, Fm:Fm + 1] * nyq_ref[...]
    for j in range(gb):
        rows = fr[j * T:(j + 1) * T, :]
        acc_ref[j, 0:T, :] = rows[:, 0:hop]
        acc_ref[j, T:, :] = jnp.zeros((ratio - 1, hop), jnp.float32)
        for k in range(1, ratio):
            acc_ref[j, k:k + T, :] += rows[:, k * hop:(k + 1) * hop]
        o_ref[0, j] = acc_ref[j, p0:p0 + nb_out, :] * inv_ref[...]


def _fused_istft(re4, im4, *, n_fft, hop, length):
    B, C, T, F = re4.shape
    assert C == 1 and F == n_fft // 2 + 1
    re = re4[:, 0]
    im = im4[:, 0]
    ratio = n_fft // hop
    start = n_fft // 2
    assert start % hop == 0 and length % hop == 0
    p0 = start // hop
    nb_out = length // hop
    A, Bm, nyq, inv_blocks = _host_tables(n_fft, n_fft, hop, T, p0, nb_out)

    gb = 2 if B % 2 == 0 else 1
    G = B // gb
    re = re.reshape(G, gb * T, F)
    im = im.reshape(G, gb * T, F)

    body = functools.partial(_fused_kernel, ratio=ratio, T=T, p0=p0,
                             nb_out=nb_out, hop=hop, gb=gb)

    def call(re_s, im_s):
        Gs = re_s.shape[0]
        return pl.pallas_call(
            body,
            out_shape=jax.ShapeDtypeStruct((Gs, gb, nb_out, hop), jnp.float32),
            grid=(Gs,),
            in_specs=[
                pl.BlockSpec((1, gb * T, F), lambda g: (g, 0, 0)),
                pl.BlockSpec((1, gb * T, F), lambda g: (g, 0, 0)),
                pl.BlockSpec((F - 1, n_fft), lambda g: (0, 0)),
                pl.BlockSpec((F - 1, n_fft), lambda g: (0, 0)),
                pl.BlockSpec((1, n_fft), lambda g: (0, 0)),
                pl.BlockSpec((nb_out, hop), lambda g: (0, 0)),
            ],
            out_specs=pl.BlockSpec((1, gb, nb_out, hop),
                                   lambda g: (g, 0, 0, 0)),
            scratch_shapes=[pltpu.VMEM((gb, T + ratio - 1, hop),
                                       jnp.float32)],
            compiler_params=pltpu.CompilerParams(
                dimension_semantics=("parallel",),
                vmem_limit_bytes=_VMEM_LIMIT,
            ),
        )(re_s, im_s, A, Bm, nyq, inv_blocks)

    n_split = 1
    Gc = G // n_split
    ys = [call(re[i * Gc:(i + 1) * Gc], im[i * Gc:(i + 1) * Gc])
          for i in range(n_split)]
    y = jnp.concatenate(ys, axis=0) if n_split > 1 else ys[0]
    return y.reshape(B, length)


def kernel(real_stft, imag_stft):
    return _fused_istft(real_stft, imag_stft,
                        n_fft=2048, hop=512, length=261632)


# speedup vs baseline: 2.0266x; 1.0022x over previous
import functools

import numpy as np

import jax
import jax.numpy as jnp
from jax.experimental import pallas as pl
from jax.experimental.pallas import tpu as pltpu


_VMEM_LIMIT = 60 * 1024 * 1024


def _hann_padded(win_length, n_fft):
    n = np.arange(win_length)
    w = 0.5 - 0.5 * np.cos(2.0 * np.pi * n / win_length)
    out = np.zeros(n_fft, dtype=np.float64)
    lpad = (n_fft - win_length) // 2
    out[lpad:lpad + win_length] = w
    return out


@functools.lru_cache(maxsize=None)
def _host_tables(n_fft, win_length, hop, T, p0, nb_out):
    F = n_fft // 2 + 1
    win = _hann_padded(win_length, n_fft)
    f = np.arange(F, dtype=np.float64)[:, None]
    o = np.arange(n_fft, dtype=np.float64)[None, :]
    ang = (2.0 * np.pi / n_fft) * f * o
    dup = np.ones((F, 1))
    dup[1:F - 1] = 2.0
    scale = win[None, :] / n_fft
    A = (dup * np.cos(ang)) * scale
    Bm = (-dup * np.sin(ang)) * scale

    win_sq = win ** 2
    n_samples = (T - 1) * hop + n_fft
    wsum = np.zeros(n_samples, dtype=np.float64)
    for t in range(T):
        wsum[t * hop:t * hop + n_fft] += win_sq
    inv = 1.0 / np.clip(wsum, 1e-11, None)
    inv_blocks = inv.reshape(-1, hop)[p0:p0 + nb_out].astype(np.float32)

    assert np.max(np.abs(Bm[F - 1])) < 1e-12
    nyq = A[F - 1:F].astype(np.float32)
    to_bf16 = lambda m: jnp.asarray(m.astype(np.float32), dtype=jnp.bfloat16)
    return (to_bf16(A[:F - 1]), to_bf16(Bm[:F - 1]),
            jnp.asarray(nyq), jnp.asarray(inv_blocks))


def _fused_kernel(re_ref, im_ref, a_ref, b_ref, nyq_ref, inv_ref, o_ref,
                  acc_ref, *, ratio, T, p0, nb_out, hop, gb):
    Fm = a_ref.shape[0]
    fr = jnp.dot(re_ref[0, :, :Fm].astype(jnp.bfloat16), a_ref[...],
                 preferred_element_type=jnp.float32)
    fr = fr + jnp.dot(im_ref[0, :, :Fm].astype(jnp.bfloat16), b_ref[...],
                      preferred_element_type=jnp.float32)
    nyq_col = re_ref[0, :, Fm:Fm + 1]
    for j in range(gb):
        rows = fr[j * T:(j + 1) * T, :]
        zc = nyq_col[j * T:(j + 1) * T, :]
        for k in range(ratio):
            tap = (rows[:, k * hop:(k + 1) * hop]
                   + zc * nyq_ref[:, k * hop:(k + 1) * hop])
            if k == 0:
                acc_ref[j, 0:T, :] = tap
                acc_ref[j, T:, :] = jnp.zeros((ratio - 1, hop), jnp.float32)
            else:
                acc_ref[j, k:k + T, :] += tap
        o_ref[0, j] = acc_ref[j, p0:p0 + nb_out, :] * inv_ref[...]


def _fused_istft(re4, im4, *, n_fft, hop, length):
    B, C, T, F = re4.shape
    assert C == 1 and F == n_fft // 2 + 1
    re = re4[:, 0]
    im = im4[:, 0]
    ratio = n_fft // hop
    start = n_fft // 2
    assert start % hop == 0 and length % hop == 0
    p0 = start // hop
    nb_out = length // hop
    A, Bm, nyq, inv_blocks = _host_tables(n_fft, n_fft, hop, T, p0, nb_out)

    gb = 2 if B % 2 == 0 else 1
    G = B // gb
    re = re.reshape(G, gb * T, F)
    im = im.reshape(G, gb * T, F)

    body = functools.partial(_fused_kernel, ratio=ratio, T=T, p0=p0,
                             nb_out=nb_out, hop=hop, gb=gb)

    def call(re_s, im_s):
        Gs = re_s.shape[0]
        return pl.pallas_call(
            body,
            out_shape=jax.ShapeDtypeStruct((Gs, gb, nb_out, hop), jnp.float32),
            grid=(Gs,),
            in_specs=[
                pl.BlockSpec((1, gb * T, F), lambda g: (g, 0, 0)),
                pl.BlockSpec((1, gb * T, F), lambda g: (g, 0, 0)),
                pl.BlockSpec((F - 1, n_fft), lambda g: (0, 0)),
                pl.BlockSpec((F - 1, n_fft), lambda g: (0, 0)),
                pl.BlockSpec((1, n_fft), lambda g: (0, 0)),
                pl.BlockSpec((nb_out, hop), lambda g: (0, 0)),
            ],
            out_specs=pl.BlockSpec((1, gb, nb_out, hop),
                                   lambda g: (g, 0, 0, 0)),
            scratch_shapes=[pltpu.VMEM((gb, T + ratio - 1, hop),
                                       jnp.float32)],
            compiler_params=pltpu.CompilerParams(
                dimension_semantics=("parallel",),
                vmem_limit_bytes=_VMEM_LIMIT,
            ),
        )(re_s, im_s, A, Bm, nyq, inv_blocks)

    n_split = 1
    Gc = G // n_split
    ys = [call(re[i * Gc:(i + 1) * Gc], im[i * Gc:(i + 1) * Gc])
          for i in range(n_split)]
    y = jnp.concatenate(ys, axis=0) if n_split > 1 else ys[0]
    return y.reshape(B, length)


def kernel(real_stft, imag_stft):
    return _fused_istft(real_stft, imag_stft,
                        n_fft=2048, hop=512, length=261632)
